# Initial kernel scaffold; baseline (speedup 1.0000x reference)
import jax
import jax.numpy as jnp
from jax.experimental import pallas as pl


def kernel(inputs, emb0, emb1, emb2):
    raise NotImplementedError("write your pallas kernel here")


# speedup vs baseline: 1.9545x; 1.9545x over previous
import functools

import jax
import jax.numpy as jnp
from jax import lax
from jax.experimental import pallas as pl
from jax.experimental.pallas import tpu as pltpu
from jax.experimental.pallas import tpu_sc as plsc

B = 8192
D = 256
E = 8192
TB = 256
TE = 1024
NW = 32
RW = B // NW
SUB = 128

def _mxu_round(q):
    return q.astype(jnp.bfloat16).astype(jnp.float32)


def _argmin_codes(r, emb_ref):
    r_sq = jnp.sum(r * r, axis=1, keepdims=True)

    def chunk(c, carry):
        bv, bi = carry
        e = emb_ref[pl.ds(c * TE, TE), :]
        d = lax.dot_general(r, e, (((1,), (1,)), ((), ())),
                            preferred_element_type=jnp.float32)
        e_sq = jnp.sum(e * e, axis=1)[None, :]
        dist = (r_sq + e_sq) - 2.0 * d
        m = jnp.min(dist, axis=1, keepdims=True)
        iota = lax.broadcasted_iota(jnp.int32, (TB, TE), 1) + c * TE
        li = jnp.min(jnp.where(dist == m, iota, jnp.int32(2 ** 30)),
                     axis=1, keepdims=True)
        better = m < bv
        return jnp.where(better, m, bv), jnp.where(better, li, bi)

    bv0 = jnp.full((TB, 1), jnp.inf, dtype=jnp.float32)
    bi0 = jnp.zeros((TB, 1), dtype=jnp.int32)
    _, bi = lax.fori_loop(0, E // TE, chunk, (bv0, bi0))
    return bi


def _head0_body(r_ref, emb_ref, idx_ref):
    idx_ref[...] = _argmin_codes(r_ref[...], emb_ref)


def _head_body(r_ref, q_ref, emb_ref, idx_ref, rout_ref):
    r = r_ref[...] - _mxu_round(q_ref[...])
    rout_ref[...] = r
    idx_ref[...] = _argmin_codes(r, emb_ref)


def _quant_body(r0_ref, r_ref, q_ref, out_ref):
    out_ref[...] = (r0_ref[...] - r_ref[...]) + _mxu_round(q_ref[...])


_RSPEC = pl.BlockSpec((TB, D), lambda i: (i, 0))
_ESPEC = pl.BlockSpec((E, D), lambda i: (0, 0))
_ISPEC = pl.BlockSpec((TB, 1), lambda i: (i, 0))
_PARAMS = pltpu.CompilerParams(dimension_semantics=("arbitrary",))


def _tc_head0(r, emb):
    return pl.pallas_call(
        _head0_body,
        grid=(B // TB,),
        in_specs=[_RSPEC, _ESPEC],
        out_specs=_ISPEC,
        out_shape=jax.ShapeDtypeStruct((B, 1), jnp.int32),
        compiler_params=_PARAMS,
    )(r, emb)


def _tc_head(r_prev, q_prev, emb):
    return pl.pallas_call(
        _head_body,
        grid=(B // TB,),
        in_specs=[_RSPEC, _RSPEC, _ESPEC],
        out_specs=(_ISPEC, _RSPEC),
        out_shape=(jax.ShapeDtypeStruct((B, 1), jnp.int32),
                   jax.ShapeDtypeStruct((B, D), jnp.float32)),
        compiler_params=_PARAMS,
    )(r_prev, q_prev, emb)


def _tc_quant(r0, r, q):
    return pl.pallas_call(
        _quant_body,
        grid=(B // TB,),
        in_specs=[_RSPEC, _RSPEC, _RSPEC],
        out_specs=_RSPEC,
        out_shape=jax.ShapeDtypeStruct((B, D), jnp.float32),
        compiler_params=_PARAMS,
    )(r0, r, q)


@functools.cache
def _sc_gather():
    mesh = plsc.VectorSubcoreMesh(core_axis_name="c", subcore_axis_name="s")

    @functools.partial(
        pl.kernel,
        out_type=jax.ShapeDtypeStruct((B, D), jnp.float32),
        mesh=mesh,
        scratch_types=[
            pltpu.VMEM((SUB,), jnp.int32),
            pltpu.VMEM((SUB, D), jnp.float32),
            pltpu.SemaphoreType.DMA,
        ],
    )
    def gather(emb_hbm, idx_hbm, out_hbm, idx_v, q_v, sem):
        base = (lax.axis_index("s") * 2 + lax.axis_index("c")) * RW
        for s in range(RW // SUB):
            b0 = base + s * SUB
            pltpu.sync_copy(idx_hbm.at[pl.ds(b0, SUB)], idx_v)
            pltpu.async_copy(emb_hbm.at[idx_v], q_v, sem).wait()
            pltpu.sync_copy(q_v, out_hbm.at[pl.ds(b0, SUB), :])

    return gather


def kernel(inputs, emb0, emb1, emb2):
    r0 = inputs[:, 0, :]
    gather = _sc_gather()

    idx0 = _tc_head0(r0, emb0)
    q0 = gather(emb0, idx0.reshape(B))
    idx1, r1 = _tc_head(r0, q0, emb1)
    q1 = gather(emb1, idx1.reshape(B))
    idx2, r2 = _tc_head(r1, q1, emb2)
    q2 = gather(emb2, idx2.reshape(B))
    quant = _tc_quant(r0, r2, q2)

    vq_codes = jnp.concatenate([idx0, idx1, idx2], axis=-1)
    return quant[:, None, :], vq_codes
